# Initial kernel scaffold; baseline (speedup 1.0000x reference)
import jax
import jax.numpy as jnp
from jax.experimental import pallas as pl


def kernel(features, sinusoids):
    raise NotImplementedError("write your pallas kernel here")


# speedup vs baseline: 1.4000x; 1.4000x over previous
import jax
import jax.numpy as jnp
from jax.experimental import pallas as pl


def _add_kernel(f_ref, s_ref, o_ref):
    o_ref[...] = f_ref[...] + s_ref[...][None]


def kernel(features, sinusoids):
    B, T, D = features.shape
    BT = 256
    grid = (B, T // BT)
    return pl.pallas_call(
        _add_kernel,
        grid=grid,
        in_specs=[
            pl.BlockSpec((1, BT, D), lambda b, j: (b, j, 0)),
            pl.BlockSpec((BT, D), lambda b, j: (j, 0)),
        ],
        out_specs=pl.BlockSpec((1, BT, D), lambda b, j: (b, j, 0)),
        out_shape=jax.ShapeDtypeStruct((B, T, D), features.dtype),
    )(features, sinusoids)
